# baseline (speedup 1.0000x reference)
import functools

import jax
import jax.numpy as jnp
from jax import lax
from jax.experimental import pallas as pl
from jax.experimental.pallas import tpu as pltpu
from jax.experimental.pallas import tpu_sc as plsc

NN = 10000
EE = 640000
EPS = 1e-5
NC, NS = 2, 16
SB = 128
BLK = 16
EP = -(-EE // (NS * BLK * SB)) * (NS * BLK * SB)
ZR = 640
W = 128
NPAD = 10240
CR = NPAD // W

R = 1000
GRID = NN // R


@functools.cache
def _make_segsum(nfc, es, want_cnt=False, dtype=jnp.float32):
    jpc = (nfc * es) // NC
    etj = EP // (es * NS)
    mesh = plsc.VectorSubcoreMesh(
        core_axis_name="c", subcore_axis_name="s",
        num_cores=NC, num_subcores=NS)

    nblk = etj // (BLK * SB)

    def impl(table, srcall, dstall, zeros, out, idx_s, idx_d, rows, acc,
             gsems, ssems, cnt_out=None):
        cid = lax.axis_index("c")
        sid = lax.axis_index("s")
        r0 = jnp.minimum(sid * ZR, NN - ZR)

        def gath(i):
            s = i % 2
            h = SB // 2
            return (
                pltpu.async_copy(table.at[idx_s.at[i, pl.ds(0, h)]],
                                 rows.at[pl.ds(s * SB, h)],
                                 gsems[2 * s]),
                pltpu.async_copy(table.at[idx_s.at[i, pl.ds(h, h)]],
                                 rows.at[pl.ds(s * SB + h, h)],
                                 gsems[2 * s + 1]),
            )

        for j in range(jpc):
            job = cid * jpc + j
            fc = job // es
            eseg = job % es
            pltpu.sync_copy(zeros.at[pl.ds(r0, ZR)], acc.at[pl.ds(r0, ZR)])
            plsc.subcore_barrier()

            def body(k, carry, fc=fc, eseg=eseg):
                base = (eseg * (EP // es // SB) + sid * (etj // SB)
                        + k * BLK)
                pltpu.sync_copy(srcall.at[pl.ds(fc * (EP // SB) + base,
                                                BLK)], idx_s)
                pltpu.sync_copy(dstall.at[pl.ds(base, BLK)], idx_d)
                gd = [None] * BLK
                sd = [None] * BLK
                gd[0] = gath(0)
                for i in range(BLK):
                    for g in gd[i]:
                        g.wait()
                    sd[i] = pltpu.async_copy(
                        rows.at[pl.ds((i % 2) * SB, SB)],
                        acc.at[idx_d.at[i]], ssems[i % 2], add=True)
                    if i + 1 < BLK:
                        if i >= 1:
                            sd[i - 1].wait()
                        gd[i + 1] = gath(i + 1)
                sd[BLK - 2].wait()
                sd[BLK - 1].wait()
                return carry

            lax.fori_loop(0, nblk, body, 0)
            plsc.subcore_barrier()
            pltpu.sync_copy(acc.at[pl.ds(r0, ZR)],
                            out.at[job, pl.ds(r0, ZR)])
            plsc.subcore_barrier()

        if cnt_out is not None:
            ones16 = jnp.full((16,), 1.0, jnp.float32)

            def fill_ones(i, carry):
                for l in range(SB // 16):
                    rows[i, pl.ds(l * 16, 16)] = ones16
                return carry

            lax.fori_loop(0, SB, fill_ones, 0)
            pltpu.sync_copy(zeros.at[pl.ds(r0, ZR)], acc.at[pl.ds(r0, ZR)])
            plsc.subcore_barrier()

            for j in range(jpc):
                eseg = cid * jpc + j

                def cbody(k, carry, eseg=eseg):
                    base = (eseg * (EP // es // SB) + sid * (etj // SB)
                            + k * BLK)
                    pltpu.sync_copy(dstall.at[pl.ds(base, BLK)], idx_d)
                    cps = [
                        pltpu.async_copy(rows.at[pl.ds(0, SB)],
                                         acc.at[idx_d.at[t]], ssems[0],
                                         add=True)
                        for t in range(BLK)
                    ]
                    for cp in cps:
                        cp.wait()
                    return carry

                lax.fori_loop(0, nblk, cbody, 0)
            plsc.subcore_barrier()
            pltpu.sync_copy(acc.at[pl.ds(r0, ZR)],
                            cnt_out.at[cid, pl.ds(r0, ZR)])

    out_type = [jax.ShapeDtypeStruct((nfc * es, NN, W), dtype)]
    scratch = [
        pltpu.VMEM((BLK, SB), jnp.int32),
        pltpu.VMEM((BLK, SB), jnp.int32),
        pltpu.VMEM((2 * SB, W), dtype),
        pltpu.VMEM_SHARED((NPAD, W), dtype),
        pltpu.SemaphoreType.DMA,
        pltpu.SemaphoreType.DMA,
        pltpu.SemaphoreType.DMA,
        pltpu.SemaphoreType.DMA,
        pltpu.SemaphoreType.DMA,
        pltpu.SemaphoreType.DMA,
    ]
    if want_cnt:
        out_type.append(jax.ShapeDtypeStruct((NC, NN, W), jnp.float32))

        def seg(table, srcall, dstall, zeros, out, cnt_out,
                idx_s, idx_d, rows, acc, g0, g1, g2, g3, sa, sb):
            impl(table, srcall, dstall, zeros, out, idx_s, idx_d, rows,
                 acc, (g0, g1, g2, g3), (sa, sb), cnt_out)
    else:
        def seg(table, srcall, dstall, zeros, out,
                idx_s, idx_d, rows, acc, g0, g1, g2, g3, sa, sb):
            impl(table, srcall, dstall, zeros, out, idx_s, idx_d, rows,
                 acc, (g0, g1, g2, g3), (sa, sb))

    return pl.kernel(seg, out_type=out_type, mesh=mesh,
                     scratch_types=scratch)


def _flatten(tab, nfc, dtype=jnp.float32):
    t = tab.astype(dtype)
    t = t.reshape(NN, nfc, W).transpose(1, 0, 2).reshape(nfc * NN, W)
    return jnp.concatenate([t, jnp.zeros((16, W), dtype)], axis=0)


def _assemble(out, nfc, es):
    s = out.reshape(nfc, es, NN, W)
    if es > 1:
        s = s.sum(axis=1)
    else:
        s = s[:, 0]
    return s.transpose(1, 0, 2).reshape(NN, nfc * W)


def _segsum(tab, srcall, dstall, nfc, es, dtype=jnp.float32):
    zeros = jnp.zeros((NN, W), dtype)
    (out,) = _make_segsum(nfc, es, dtype=dtype)(
        _flatten(tab, nfc, dtype), srcall, dstall, zeros)
    return _assemble(out.astype(jnp.float32), nfc, es)


@functools.cache
def _make_fused(din, dout):
    def body(s_ref, y_ref, ic_ref, a_ref, c_ref, w_ref, b_ref,
             out_ref, st_ref):
        i = pl.program_id(0)
        a = a_ref[...]
        c = c_ref[...]
        hn = y_ref[...] * a + c
        an = s_ref[...] * ic_ref[...] * a + c
        z = jnp.dot(an, w_ref[0:din, :], preferred_element_type=jnp.float32)
        z = z + jnp.dot(hn, w_ref[din:2 * din, :],
                        preferred_element_type=jnp.float32)
        z = z + b_ref[...]
        y = jnp.maximum(z, 0.0)
        out_ref[...] = y

        @pl.when(i == 0)
        def _():
            st_ref[...] = jnp.zeros_like(st_ref)

        st_ref[...] += jnp.concatenate(
            [jnp.sum(y, axis=0, keepdims=True),
             jnp.sum(y * y, axis=0, keepdims=True)], axis=0)

    return pl.pallas_call(
        body,
        grid=(GRID,),
        in_specs=[
            pl.BlockSpec((R, din), lambda i: (i, 0)),
            pl.BlockSpec((R, din), lambda i: (i, 0)),
            pl.BlockSpec((R, 1), lambda i: (i, 0)),
            pl.BlockSpec((1, din), lambda i: (0, 0)),
            pl.BlockSpec((1, din), lambda i: (0, 0)),
            pl.BlockSpec((2 * din, dout), lambda i: (0, 0)),
            pl.BlockSpec((1, dout), lambda i: (0, 0)),
        ],
        out_specs=[
            pl.BlockSpec((R, dout), lambda i: (i, 0)),
            pl.BlockSpec((2, dout), lambda i: (0, 0)),
        ],
        out_shape=[
            jax.ShapeDtypeStruct((NN, dout), jnp.float32),
            jax.ShapeDtypeStruct((2, dout), jnp.float32),
        ],
    )


@functools.cache
def _make_affmm(din, dout):
    def body(y_ref, a_ref, c_ref, w_ref, out_ref):
        hn = y_ref[...] * a_ref[...] + c_ref[...]
        out_ref[...] = jnp.dot(hn, w_ref[...],
                               preferred_element_type=jnp.float32)

    return pl.pallas_call(
        body,
        grid=(GRID,),
        in_specs=[
            pl.BlockSpec((R, din), lambda i: (i, 0)),
            pl.BlockSpec((1, din), lambda i: (0, 0)),
            pl.BlockSpec((1, din), lambda i: (0, 0)),
            pl.BlockSpec((din, dout), lambda i: (0, 0)),
        ],
        out_specs=pl.BlockSpec((R, dout), lambda i: (i, 0)),
        out_shape=jax.ShapeDtypeStruct((NN, dout), jnp.float32),
    )


@functools.cache
def _make_final(din, dout):
    def body(s_ref, y_ref, ic_ref, a_ref, c_ref, w_ref, b_ref, out_ref):
        hn = y_ref[...] * a_ref[...] + c_ref[...]
        z = s_ref[...] * ic_ref[...]
        z = z + jnp.dot(hn, w_ref[...], preferred_element_type=jnp.float32)
        z = z + b_ref[...]
        m = jnp.max(z, axis=1, keepdims=True)
        ez = jnp.exp(z - m)
        out_ref[...] = z - m - jnp.log(jnp.sum(ez, axis=1, keepdims=True))

    return pl.pallas_call(
        body,
        grid=(GRID,),
        in_specs=[
            pl.BlockSpec((R, dout), lambda i: (i, 0)),
            pl.BlockSpec((R, din), lambda i: (i, 0)),
            pl.BlockSpec((R, 1), lambda i: (i, 0)),
            pl.BlockSpec((1, din), lambda i: (0, 0)),
            pl.BlockSpec((1, din), lambda i: (0, 0)),
            pl.BlockSpec((din, dout), lambda i: (0, 0)),
            pl.BlockSpec((1, dout), lambda i: (0, 0)),
        ],
        out_specs=pl.BlockSpec((R, dout), lambda i: (i, 0)),
        out_shape=jax.ShapeDtypeStruct((NN, dout), jnp.float32),
    )


_DIMS = [128, 128, 256, 512, 1024, 40]
_CFG = {1: (1, 4), 2: (1, 4), 3: (2, 1), 4: (4, 1)}


def kernel(x, edge_index, params):
    src = edge_index[0]
    dst = edge_index[1]
    pad = EP - EE
    srcp = jnp.concatenate([src, jnp.zeros((pad,), jnp.int32)])
    dstp = jnp.concatenate([dst, jnp.full((pad,), NN, jnp.int32)])
    ispad = jnp.arange(EP, dtype=jnp.int32) >= EE

    def mk_srcall(nfc):
        offs = (jnp.arange(nfc, dtype=jnp.int32) * NN)[:, None]
        sa = jnp.where(ispad[None, :], nfc * NN, srcp[None, :] + offs)
        return sa.reshape(nfc * EP // SB, SB)

    srcall = {nfc: mk_srcall(nfc) for nfc in (1, 2, 4)}
    dst2d = dstp.reshape(EP // SB, SB)
    zeros = jnp.zeros((NN, W), jnp.float32)

    s1out, cnt_out = _make_segsum(1, 4, True)(
        _flatten(x, 1), srcall[1], dst2d, zeros)
    cnt = cnt_out[0, :, 0] + cnt_out[1, :, 0]
    invc = (1.0 / jnp.maximum(cnt, 1.0))[:, None]

    a = jnp.ones((1, _DIMS[0]), jnp.float32)
    c = jnp.zeros((1, _DIMS[0]), jnp.float32)
    Y = x
    for i in range(1, 5):
        din, dout = _DIMS[i - 1], _DIMS[i]
        nfc, es = _CFG[i]
        if i == 1:
            S = _assemble(s1out, 1, 4)
        else:
            S = _segsum(Y, srcall[nfc], dst2d, nfc, es)
        wcat = jnp.concatenate([params[f"Wl{i}"], params[f"Wr{i}"]], axis=0)
        Y, st = _make_fused(din, dout)(
            S, Y, invc, a, c, wcat, params[f"b{i}"][None, :])
        mu = st[0] / NN
        var = st[1] / NN - mu * mu
        ai = params[f"g{i}"] / jnp.sqrt(var + EPS)
        ci = params[f"be{i}"] - mu * ai
        a, c = ai[None, :], ci[None, :]

    t = _make_affmm(1024, 40)(Y, a, c, params["Wl5"])
    tpad = jnp.pad(t, ((0, 0), (0, W - 40)))
    S5 = _segsum(tpad, srcall[1], dst2d, 1, 4)[:, :40]
    return _make_final(1024, 40)(
        S5, Y, invc, a, c, params["Wr5"], params["b5"][None, :])


# speedup vs baseline: 1.0475x; 1.0475x over previous
import functools

import jax
import jax.numpy as jnp
from jax import lax
from jax.experimental import pallas as pl
from jax.experimental.pallas import tpu as pltpu
from jax.experimental.pallas import tpu_sc as plsc

NN = 10000
EE = 640000
EPS = 1e-5
NC, NS = 2, 16
SB = 128
BLK = 16
EP = -(-EE // (NS * BLK * SB)) * (NS * BLK * SB)
ZR = 640
W = 128
NPAD = 10240
CR = NPAD // W

R = 1000
GRID = NN // R


@functools.cache
def _make_segsum(nfc, es, want_cnt=False, dtype=jnp.float32, dup=False):
    jpc = (nfc * es) // NC
    etj = EP // (es * NS)
    mesh = plsc.VectorSubcoreMesh(
        core_axis_name="c", subcore_axis_name="s",
        num_cores=NC, num_subcores=NS)

    nblk = etj // (BLK * SB)

    def impl(table, srcall, dstall, zeros, out, idx_s, idx_d, rows, acc,
             gsems, ssems, cnt_out=None):
        cid = lax.axis_index("c")
        sid = lax.axis_index("s")
        r0 = jnp.minimum(sid * ZR, NN - ZR)

        def gath(i):
            s = i % 2
            h = SB // 2
            return (
                pltpu.async_copy(table.at[idx_s.at[i, pl.ds(0, h)]],
                                 rows.at[pl.ds(s * SB, h)],
                                 gsems[2 * s]),
                pltpu.async_copy(table.at[idx_s.at[i, pl.ds(h, h)]],
                                 rows.at[pl.ds(s * SB + h, h)],
                                 gsems[2 * s + 1]),
            )

        for j in range(jpc):
            job = cid * jpc + j
            fc = job // es
            eseg = job % es
            pltpu.sync_copy(zeros.at[pl.ds(r0, ZR)], acc.at[pl.ds(r0, ZR)])
            plsc.subcore_barrier()

            goff = job if dup else fc

            def body(k, carry, goff=goff, eseg=eseg):
                base = (eseg * (EP // es // SB) + sid * (etj // SB)
                        + k * BLK)
                pltpu.sync_copy(srcall.at[pl.ds(goff * (EP // SB) + base,
                                                BLK)], idx_s)
                pltpu.sync_copy(dstall.at[pl.ds(base, BLK)], idx_d)
                gd = [None] * BLK
                sd = [None] * BLK
                gd[0] = gath(0)
                for i in range(BLK):
                    for g in gd[i]:
                        g.wait()
                    sd[i] = pltpu.async_copy(
                        rows.at[pl.ds((i % 2) * SB, SB)],
                        acc.at[idx_d.at[i]], ssems[i % 2], add=True)
                    if i + 1 < BLK:
                        if i >= 1:
                            sd[i - 1].wait()
                        gd[i + 1] = gath(i + 1)
                sd[BLK - 2].wait()
                sd[BLK - 1].wait()
                return carry

            lax.fori_loop(0, nblk, body, 0)
            plsc.subcore_barrier()
            pltpu.sync_copy(acc.at[pl.ds(r0, ZR)],
                            out.at[job, pl.ds(r0, ZR)])
            plsc.subcore_barrier()

        if cnt_out is not None:
            ones16 = jnp.full((16,), 1.0, jnp.float32)

            def fill_ones(i, carry):
                for l in range(SB // 16):
                    rows[i, pl.ds(l * 16, 16)] = ones16
                return carry

            lax.fori_loop(0, SB, fill_ones, 0)
            pltpu.sync_copy(zeros.at[pl.ds(r0, ZR)], acc.at[pl.ds(r0, ZR)])
            plsc.subcore_barrier()

            for j in range(jpc):
                eseg = cid * jpc + j

                def cbody(k, carry, eseg=eseg):
                    base = (eseg * (EP // es // SB) + sid * (etj // SB)
                            + k * BLK)
                    pltpu.sync_copy(dstall.at[pl.ds(base, BLK)], idx_d)
                    cps = [
                        pltpu.async_copy(rows.at[pl.ds(0, SB)],
                                         acc.at[idx_d.at[t]], ssems[0],
                                         add=True)
                        for t in range(BLK)
                    ]
                    for cp in cps:
                        cp.wait()
                    return carry

                lax.fori_loop(0, nblk, cbody, 0)
            plsc.subcore_barrier()
            pltpu.sync_copy(acc.at[pl.ds(r0, ZR)],
                            cnt_out.at[cid, pl.ds(r0, ZR)])

    out_type = [jax.ShapeDtypeStruct((nfc * es, NN, W), dtype)]
    scratch = [
        pltpu.VMEM((BLK, SB), jnp.int32),
        pltpu.VMEM((BLK, SB), jnp.int32),
        pltpu.VMEM((2 * SB, W), dtype),
        pltpu.VMEM_SHARED((NPAD, W), dtype),
        pltpu.SemaphoreType.DMA,
        pltpu.SemaphoreType.DMA,
        pltpu.SemaphoreType.DMA,
        pltpu.SemaphoreType.DMA,
        pltpu.SemaphoreType.DMA,
        pltpu.SemaphoreType.DMA,
    ]
    if want_cnt:
        out_type.append(jax.ShapeDtypeStruct((NC, NN, W), jnp.float32))

        def seg(table, srcall, dstall, zeros, out, cnt_out,
                idx_s, idx_d, rows, acc, g0, g1, g2, g3, sa, sb):
            impl(table, srcall, dstall, zeros, out, idx_s, idx_d, rows,
                 acc, (g0, g1, g2, g3), (sa, sb), cnt_out)
    else:
        def seg(table, srcall, dstall, zeros, out,
                idx_s, idx_d, rows, acc, g0, g1, g2, g3, sa, sb):
            impl(table, srcall, dstall, zeros, out, idx_s, idx_d, rows,
                 acc, (g0, g1, g2, g3), (sa, sb))

    return pl.kernel(seg, out_type=out_type, mesh=mesh,
                     scratch_types=scratch)


def _flatten(tab, nfc, dtype=jnp.float32):
    t = tab.astype(dtype)
    t = t.reshape(NN, nfc, W).transpose(1, 0, 2).reshape(nfc * NN, W)
    return jnp.concatenate([t, jnp.zeros((16, W), dtype)], axis=0)


def _assemble(out, nfc, es):
    s = out.reshape(nfc, es, NN, W)
    if es > 1:
        s = s.sum(axis=1)
    else:
        s = s[:, 0]
    return s.transpose(1, 0, 2).reshape(NN, nfc * W)


def _segsum_dup(tab, srcall2, dstall):
    zeros = jnp.zeros((NN, W), jnp.float32)
    (out,) = _make_segsum(1, 2, dup=True)(
        _flatten(jnp.tile(tab, (1, 2)), 2), srcall2, dstall, zeros)
    return _assemble(out, 1, 2)


def _segsum(tab, srcall, dstall, nfc, es, dtype=jnp.float32):
    zeros = jnp.zeros((NN, W), dtype)
    (out,) = _make_segsum(nfc, es, dtype=dtype)(
        _flatten(tab, nfc, dtype), srcall, dstall, zeros)
    return _assemble(out.astype(jnp.float32), nfc, es)


@functools.cache
def _make_fused(din, dout):
    def body(s_ref, y_ref, ic_ref, a_ref, c_ref, w_ref, b_ref,
             out_ref, st_ref):
        i = pl.program_id(0)
        a = a_ref[...]
        c = c_ref[...]
        hn = y_ref[...] * a + c
        an = s_ref[...] * ic_ref[...] * a + c
        z = jnp.dot(an, w_ref[0:din, :], preferred_element_type=jnp.float32)
        z = z + jnp.dot(hn, w_ref[din:2 * din, :],
                        preferred_element_type=jnp.float32)
        z = z + b_ref[...]
        y = jnp.maximum(z, 0.0)
        out_ref[...] = y

        @pl.when(i == 0)
        def _():
            st_ref[...] = jnp.zeros_like(st_ref)

        st_ref[...] += jnp.concatenate(
            [jnp.sum(y, axis=0, keepdims=True),
             jnp.sum(y * y, axis=0, keepdims=True)], axis=0)

    return pl.pallas_call(
        body,
        grid=(GRID,),
        in_specs=[
            pl.BlockSpec((R, din), lambda i: (i, 0)),
            pl.BlockSpec((R, din), lambda i: (i, 0)),
            pl.BlockSpec((R, 1), lambda i: (i, 0)),
            pl.BlockSpec((1, din), lambda i: (0, 0)),
            pl.BlockSpec((1, din), lambda i: (0, 0)),
            pl.BlockSpec((2 * din, dout), lambda i: (0, 0)),
            pl.BlockSpec((1, dout), lambda i: (0, 0)),
        ],
        out_specs=[
            pl.BlockSpec((R, dout), lambda i: (i, 0)),
            pl.BlockSpec((2, dout), lambda i: (0, 0)),
        ],
        out_shape=[
            jax.ShapeDtypeStruct((NN, dout), jnp.float32),
            jax.ShapeDtypeStruct((2, dout), jnp.float32),
        ],
    )


@functools.cache
def _make_affmm(din, dout):
    def body(y_ref, a_ref, c_ref, w_ref, out_ref):
        hn = y_ref[...] * a_ref[...] + c_ref[...]
        out_ref[...] = jnp.dot(hn, w_ref[...],
                               preferred_element_type=jnp.float32)

    return pl.pallas_call(
        body,
        grid=(GRID,),
        in_specs=[
            pl.BlockSpec((R, din), lambda i: (i, 0)),
            pl.BlockSpec((1, din), lambda i: (0, 0)),
            pl.BlockSpec((1, din), lambda i: (0, 0)),
            pl.BlockSpec((din, dout), lambda i: (0, 0)),
        ],
        out_specs=pl.BlockSpec((R, dout), lambda i: (i, 0)),
        out_shape=jax.ShapeDtypeStruct((NN, dout), jnp.float32),
    )


@functools.cache
def _make_final(din, dout):
    def body(s_ref, y_ref, ic_ref, a_ref, c_ref, w_ref, b_ref, out_ref):
        hn = y_ref[...] * a_ref[...] + c_ref[...]
        z = s_ref[...] * ic_ref[...]
        z = z + jnp.dot(hn, w_ref[...], preferred_element_type=jnp.float32)
        z = z + b_ref[...]
        m = jnp.max(z, axis=1, keepdims=True)
        ez = jnp.exp(z - m)
        out_ref[...] = z - m - jnp.log(jnp.sum(ez, axis=1, keepdims=True))

    return pl.pallas_call(
        body,
        grid=(GRID,),
        in_specs=[
            pl.BlockSpec((R, dout), lambda i: (i, 0)),
            pl.BlockSpec((R, din), lambda i: (i, 0)),
            pl.BlockSpec((R, 1), lambda i: (i, 0)),
            pl.BlockSpec((1, din), lambda i: (0, 0)),
            pl.BlockSpec((1, din), lambda i: (0, 0)),
            pl.BlockSpec((din, dout), lambda i: (0, 0)),
            pl.BlockSpec((1, dout), lambda i: (0, 0)),
        ],
        out_specs=pl.BlockSpec((R, dout), lambda i: (i, 0)),
        out_shape=jax.ShapeDtypeStruct((NN, dout), jnp.float32),
    )


_DIMS = [128, 128, 256, 512, 1024, 40]
_CFG = {1: (1, 2), 2: (1, 2), 3: (2, 1), 4: (4, 1)}


def kernel(x, edge_index, params):
    src = edge_index[0]
    dst = edge_index[1]
    pad = EP - EE
    srcp = jnp.concatenate([src, jnp.zeros((pad,), jnp.int32)])
    dstp = jnp.concatenate([dst, jnp.full((pad,), NN, jnp.int32)])
    ispad = jnp.arange(EP, dtype=jnp.int32) >= EE

    def mk_srcall(nfc):
        offs = (jnp.arange(nfc, dtype=jnp.int32) * NN)[:, None]
        sa = jnp.where(ispad[None, :], nfc * NN, srcp[None, :] + offs)
        return sa.reshape(nfc * EP // SB, SB)

    srcall = {nfc: mk_srcall(nfc) for nfc in (1, 2, 4)}
    dst2d = dstp.reshape(EP // SB, SB)
    zeros = jnp.zeros((NN, W), jnp.float32)

    s1out, cnt_out = _make_segsum(1, 2, True, dup=True)(
        _flatten(jnp.tile(x, (1, 2)), 2), srcall[2], dst2d, zeros)
    cnt = cnt_out[0, :, 0] + cnt_out[1, :, 0]
    invc = (1.0 / jnp.maximum(cnt, 1.0))[:, None]

    a = jnp.ones((1, _DIMS[0]), jnp.float32)
    c = jnp.zeros((1, _DIMS[0]), jnp.float32)
    Y = x
    for i in range(1, 5):
        din, dout = _DIMS[i - 1], _DIMS[i]
        nfc, es = _CFG[i]
        if i == 1:
            S = _assemble(s1out, 1, 2)
        elif nfc == 1:
            S = _segsum_dup(Y, srcall[2], dst2d)
        else:
            S = _segsum(Y, srcall[nfc], dst2d, nfc, es)
        wcat = jnp.concatenate([params[f"Wl{i}"], params[f"Wr{i}"]], axis=0)
        Y, st = _make_fused(din, dout)(
            S, Y, invc, a, c, wcat, params[f"b{i}"][None, :])
        mu = st[0] / NN
        var = st[1] / NN - mu * mu
        ai = params[f"g{i}"] / jnp.sqrt(var + EPS)
        ci = params[f"be{i}"] - mu * ai
        a, c = ai[None, :], ci[None, :]

    t = _make_affmm(1024, 40)(Y, a, c, params["Wl5"])
    tpad = jnp.pad(t, ((0, 0), (0, W - 40)))
    S5 = _segsum_dup(tpad, srcall[2], dst2d)[:, :40]
    return _make_final(1024, 40)(
        S5, Y, invc, a, c, params["Wr5"], params["b5"][None, :])


# speedup vs baseline: 1.0952x; 1.0456x over previous
import functools

import jax
import jax.numpy as jnp
from jax import lax
from jax.experimental import pallas as pl
from jax.experimental.pallas import tpu as pltpu
from jax.experimental.pallas import tpu_sc as plsc

NN = 10000
EE = 640000
EPS = 1e-5
NC, NS = 2, 16
SB = 128
BLK = 16
EP = -(-EE // (NS * BLK * SB)) * (NS * BLK * SB)
ZR = 640
W = 128
NPAD = 10240
CR = NPAD // W

R = 1000
GRID = NN // R


@functools.cache
def _make_segsum(nfc, es, want_cnt=False, dtype=jnp.float32, dup=False):
    jpc = (nfc * es) // NC
    etj = EP // (es * NS)
    mesh = plsc.VectorSubcoreMesh(
        core_axis_name="c", subcore_axis_name="s",
        num_cores=NC, num_subcores=NS)

    nblk = etj // (BLK * SB)

    def impl(table, srcall, dstall, zeros, out, idx_s, idx_d, rows, acc,
             gsems, ssems, cnt_out=None):
        cid = lax.axis_index("c")
        sid = lax.axis_index("s")
        r0 = jnp.minimum(sid * ZR, NN - ZR)

        def gath(i):
            s = i % 2
            h = SB // 2
            return (
                pltpu.async_copy(table.at[idx_s.at[i, pl.ds(0, h)]],
                                 rows.at[pl.ds(s * SB, h)],
                                 gsems[2 * s]),
                pltpu.async_copy(table.at[idx_s.at[i, pl.ds(h, h)]],
                                 rows.at[pl.ds(s * SB + h, h)],
                                 gsems[2 * s + 1]),
            )

        for j in range(jpc):
            job = cid * jpc + j
            fc = job // es
            eseg = job % es
            pltpu.sync_copy(zeros.at[pl.ds(r0, ZR)], acc.at[pl.ds(r0, ZR)])
            plsc.subcore_barrier()

            goff = job if dup else fc

            def body(k, carry, goff=goff, eseg=eseg):
                base = (sid * (etj // SB) * es + (k * es + eseg) * BLK)
                pltpu.sync_copy(srcall.at[pl.ds(goff * (EP // SB) + base,
                                                BLK)], idx_s)
                pltpu.sync_copy(dstall.at[pl.ds(base, BLK)], idx_d)
                gd = [None] * BLK
                sd = [None] * BLK
                gd[0] = gath(0)
                for i in range(BLK):
                    for g in gd[i]:
                        g.wait()
                    sd[i] = pltpu.async_copy(
                        rows.at[pl.ds((i % 2) * SB, SB)],
                        acc.at[idx_d.at[i]], ssems[i % 2], add=True)
                    if i + 1 < BLK:
                        if i >= 1:
                            sd[i - 1].wait()
                        gd[i + 1] = gath(i + 1)
                sd[BLK - 2].wait()
                sd[BLK - 1].wait()
                return carry

            lax.fori_loop(0, nblk, body, 0)
            plsc.subcore_barrier()
            pltpu.sync_copy(acc.at[pl.ds(r0, ZR)],
                            out.at[job, pl.ds(r0, ZR)])
            plsc.subcore_barrier()

        if cnt_out is not None:
            ones16 = jnp.full((16,), 1.0, jnp.float32)

            def fill_ones(i, carry):
                for l in range(SB // 16):
                    rows[i, pl.ds(l * 16, 16)] = ones16
                return carry

            lax.fori_loop(0, SB, fill_ones, 0)
            pltpu.sync_copy(zeros.at[pl.ds(r0, ZR)], acc.at[pl.ds(r0, ZR)])
            plsc.subcore_barrier()

            for j in range(jpc):
                eseg = cid * jpc + j

                def cbody(k, carry, eseg=eseg):
                    base = (sid * (etj // SB) * es + (k * es + eseg) * BLK)
                    pltpu.sync_copy(dstall.at[pl.ds(base, BLK)], idx_d)
                    cps = [
                        pltpu.async_copy(rows.at[pl.ds(0, SB)],
                                         acc.at[idx_d.at[t]], ssems[0],
                                         add=True)
                        for t in range(BLK)
                    ]
                    for cp in cps:
                        cp.wait()
                    return carry

                lax.fori_loop(0, nblk, cbody, 0)
            plsc.subcore_barrier()
            pltpu.sync_copy(acc.at[pl.ds(r0, ZR)],
                            cnt_out.at[cid, pl.ds(r0, ZR)])

    out_type = [jax.ShapeDtypeStruct((nfc * es, NN, W), dtype)]
    scratch = [
        pltpu.VMEM((BLK, SB), jnp.int32),
        pltpu.VMEM((BLK, SB), jnp.int32),
        pltpu.VMEM((2 * SB, W), dtype),
        pltpu.VMEM_SHARED((NPAD, W), dtype),
        pltpu.SemaphoreType.DMA,
        pltpu.SemaphoreType.DMA,
        pltpu.SemaphoreType.DMA,
        pltpu.SemaphoreType.DMA,
        pltpu.SemaphoreType.DMA,
        pltpu.SemaphoreType.DMA,
    ]
    if want_cnt:
        out_type.append(jax.ShapeDtypeStruct((NC, NN, W), jnp.float32))

        def seg(table, srcall, dstall, zeros, out, cnt_out,
                idx_s, idx_d, rows, acc, g0, g1, g2, g3, sa, sb):
            impl(table, srcall, dstall, zeros, out, idx_s, idx_d, rows,
                 acc, (g0, g1, g2, g3), (sa, sb), cnt_out)
    else:
        def seg(table, srcall, dstall, zeros, out,
                idx_s, idx_d, rows, acc, g0, g1, g2, g3, sa, sb):
            impl(table, srcall, dstall, zeros, out, idx_s, idx_d, rows,
                 acc, (g0, g1, g2, g3), (sa, sb))

    return pl.kernel(seg, out_type=out_type, mesh=mesh,
                     scratch_types=scratch)


def _flatten(tab, nfc, dtype=jnp.float32):
    t = tab.astype(dtype)
    t = t.reshape(NN, nfc, W).transpose(1, 0, 2).reshape(nfc * NN, W)
    return jnp.concatenate([t, jnp.zeros((16, W), dtype)], axis=0)


def _assemble(out, nfc, es):
    s = out.reshape(nfc, es, NN, W)
    if es > 1:
        s = s.sum(axis=1)
    else:
        s = s[:, 0]
    return s.transpose(1, 0, 2).reshape(NN, nfc * W)


def _segsum_dup(tab, srcall2, dstall):
    zeros = jnp.zeros((NN, W), jnp.float32)
    (out,) = _make_segsum(1, 2, dup=True)(
        _flatten(jnp.tile(tab, (1, 2)), 2), srcall2, dstall, zeros)
    return _assemble(out, 1, 2)


def _segsum(tab, srcall, dstall, nfc, es, dtype=jnp.float32):
    zeros = jnp.zeros((NN, W), dtype)
    (out,) = _make_segsum(nfc, es, dtype=dtype)(
        _flatten(tab, nfc, dtype), srcall, dstall, zeros)
    return _assemble(out.astype(jnp.float32), nfc, es)


@functools.cache
def _make_fused(din, dout):
    def body(s_ref, y_ref, ic_ref, a_ref, c_ref, w_ref, b_ref,
             out_ref, st_ref):
        i = pl.program_id(0)
        a = a_ref[...]
        c = c_ref[...]
        hn = y_ref[...] * a + c
        an = s_ref[...] * ic_ref[...] * a + c
        z = jnp.dot(an, w_ref[0:din, :], preferred_element_type=jnp.float32)
        z = z + jnp.dot(hn, w_ref[din:2 * din, :],
                        preferred_element_type=jnp.float32)
        z = z + b_ref[...]
        y = jnp.maximum(z, 0.0)
        out_ref[...] = y

        @pl.when(i == 0)
        def _():
            st_ref[...] = jnp.zeros_like(st_ref)

        st_ref[...] += jnp.concatenate(
            [jnp.sum(y, axis=0, keepdims=True),
             jnp.sum(y * y, axis=0, keepdims=True)], axis=0)

    return pl.pallas_call(
        body,
        grid=(GRID,),
        in_specs=[
            pl.BlockSpec((R, din), lambda i: (i, 0)),
            pl.BlockSpec((R, din), lambda i: (i, 0)),
            pl.BlockSpec((R, 1), lambda i: (i, 0)),
            pl.BlockSpec((1, din), lambda i: (0, 0)),
            pl.BlockSpec((1, din), lambda i: (0, 0)),
            pl.BlockSpec((2 * din, dout), lambda i: (0, 0)),
            pl.BlockSpec((1, dout), lambda i: (0, 0)),
        ],
        out_specs=[
            pl.BlockSpec((R, dout), lambda i: (i, 0)),
            pl.BlockSpec((2, dout), lambda i: (0, 0)),
        ],
        out_shape=[
            jax.ShapeDtypeStruct((NN, dout), jnp.float32),
            jax.ShapeDtypeStruct((2, dout), jnp.float32),
        ],
    )


@functools.cache
def _make_affmm(din, dout):
    def body(y_ref, a_ref, c_ref, w_ref, out_ref):
        hn = y_ref[...] * a_ref[...] + c_ref[...]
        out_ref[...] = jnp.dot(hn, w_ref[...],
                               preferred_element_type=jnp.float32)

    return pl.pallas_call(
        body,
        grid=(GRID,),
        in_specs=[
            pl.BlockSpec((R, din), lambda i: (i, 0)),
            pl.BlockSpec((1, din), lambda i: (0, 0)),
            pl.BlockSpec((1, din), lambda i: (0, 0)),
            pl.BlockSpec((din, dout), lambda i: (0, 0)),
        ],
        out_specs=pl.BlockSpec((R, dout), lambda i: (i, 0)),
        out_shape=jax.ShapeDtypeStruct((NN, dout), jnp.float32),
    )


@functools.cache
def _make_final(din, dout):
    def body(s_ref, y_ref, ic_ref, a_ref, c_ref, w_ref, b_ref, out_ref):
        hn = y_ref[...] * a_ref[...] + c_ref[...]
        z = s_ref[...] * ic_ref[...]
        z = z + jnp.dot(hn, w_ref[...], preferred_element_type=jnp.float32)
        z = z + b_ref[...]
        m = jnp.max(z, axis=1, keepdims=True)
        ez = jnp.exp(z - m)
        out_ref[...] = z - m - jnp.log(jnp.sum(ez, axis=1, keepdims=True))

    return pl.pallas_call(
        body,
        grid=(GRID,),
        in_specs=[
            pl.BlockSpec((R, dout), lambda i: (i, 0)),
            pl.BlockSpec((R, din), lambda i: (i, 0)),
            pl.BlockSpec((R, 1), lambda i: (i, 0)),
            pl.BlockSpec((1, din), lambda i: (0, 0)),
            pl.BlockSpec((1, din), lambda i: (0, 0)),
            pl.BlockSpec((din, dout), lambda i: (0, 0)),
            pl.BlockSpec((1, dout), lambda i: (0, 0)),
        ],
        out_specs=pl.BlockSpec((R, dout), lambda i: (i, 0)),
        out_shape=jax.ShapeDtypeStruct((NN, dout), jnp.float32),
    )


_DIMS = [128, 128, 256, 512, 1024, 40]
_CFG = {1: (1, 2), 2: (1, 2), 3: (2, 1), 4: (4, 1)}


def kernel(x, edge_index, params):
    src = edge_index[0]
    dst = edge_index[1]
    pad = EP - EE
    srcp = jnp.concatenate([src, jnp.zeros((pad,), jnp.int32)])
    dstp = jnp.concatenate([dst, jnp.full((pad,), NN, jnp.int32)])
    ispad = jnp.arange(EP, dtype=jnp.int32) >= EE

    def mk_srcall(nfc):
        offs = (jnp.arange(nfc, dtype=jnp.int32) * NN)[:, None]
        sa = jnp.where(ispad[None, :], nfc * NN, srcp[None, :] + offs)
        return sa.reshape(nfc * EP // SB, SB)

    srcall = {nfc: mk_srcall(nfc) for nfc in (1, 2, 4)}
    dst2d = dstp.reshape(EP // SB, SB)
    zeros = jnp.zeros((NN, W), jnp.float32)

    s1out, cnt_out = _make_segsum(1, 2, True, dup=True)(
        _flatten(jnp.tile(x, (1, 2)), 2), srcall[2], dst2d, zeros)
    cnt = cnt_out[0, :, 0] + cnt_out[1, :, 0]
    invc = (1.0 / jnp.maximum(cnt, 1.0))[:, None]

    a = jnp.ones((1, _DIMS[0]), jnp.float32)
    c = jnp.zeros((1, _DIMS[0]), jnp.float32)
    Y = x
    for i in range(1, 5):
        din, dout = _DIMS[i - 1], _DIMS[i]
        nfc, es = _CFG[i]
        if i == 1:
            S = _assemble(s1out, 1, 2)
        elif nfc == 1:
            S = _segsum_dup(Y, srcall[2], dst2d)
        else:
            S = _segsum(Y, srcall[nfc], dst2d, nfc, es)
        wcat = jnp.concatenate([params[f"Wl{i}"], params[f"Wr{i}"]], axis=0)
        Y, st = _make_fused(din, dout)(
            S, Y, invc, a, c, wcat, params[f"b{i}"][None, :])
        mu = st[0] / NN
        var = st[1] / NN - mu * mu
        ai = params[f"g{i}"] / jnp.sqrt(var + EPS)
        ci = params[f"be{i}"] - mu * ai
        a, c = ai[None, :], ci[None, :]

    t = _make_affmm(1024, 40)(Y, a, c, params["Wl5"])
    tpad = jnp.pad(t, ((0, 0), (0, W - 40)))
    S5 = _segsum_dup(tpad, srcall[2], dst2d)[:, :40]
    return _make_final(1024, 40)(
        S5, Y, invc, a, c, params["Wr5"], params["b5"][None, :])
